# Initial kernel scaffold; baseline (speedup 1.0000x reference)
import jax
import jax.numpy as jnp
from jax.experimental import pallas as pl


def kernel(x, edge_index, edge_weight, Wz, bz, Wr, br, Wh, bh, LzW, Lzb, LrW, Lrb, LhW, Lhb, att, linW, linb):
    raise NotImplementedError("write your pallas kernel here")


# speedup vs baseline: 8.0121x; 8.0121x over previous
import functools

import jax
import jax.numpy as jnp
from jax import lax
from jax.experimental import pallas as pl
from jax.experimental.pallas import tpu as pltpu
from jax.experimental.pallas import tpu_sc as plsc

N_NODES = 100000
F_IN = 8
F_OUT = 24
PERIODS = 12

NTILES = 16
NCORES = 2
CH = 512
FB = 16
NBLK = PERIODS * F_IN // FB
BPC = NBLK // NCORES

NP_PAD = 100096
STRIPE = NP_PAD // NTILES
E_TOT = 1600000 + N_NODES
PAIRS = 104
E_PAD = NTILES * 2 * CH * PAIRS
PAIRS_HALF = PAIRS // 2

_mesh = plsc.VectorSubcoreMesh(core_axis_name="c", subcore_axis_name="s")


def _rsqrt_newton(d):
    bits = lax.bitcast_convert_type(d, jnp.int32)
    y = lax.bitcast_convert_type(jnp.int32(0x5F3759DF) - (bits >> 1),
                                 jnp.float32)
    for _ in range(3):
        y = y * (1.5 - 0.5 * d * y * y)
    return y


@functools.partial(
    pl.kernel,
    out_type=jax.ShapeDtypeStruct((E_PAD,), jnp.float32),
    mesh=_mesh,
    scratch_types=[
        pltpu.VMEM_SHARED((NP_PAD,), jnp.float32),
        pltpu.VMEM((CH,), jnp.int32),
        pltpu.VMEM((CH,), jnp.int32),
        pltpu.VMEM((CH,), jnp.float32),
        pltpu.VMEM((CH,), jnp.float32),
        pltpu.VMEM((CH,), jnp.float32),
        pltpu.VMEM((CH,), jnp.float32),
        pltpu.VMEM((STRIPE,), jnp.float32),
        pltpu.SemaphoreType.DMA,
    ],
)
def _norm_kernel(row_hbm, col_hbm, w_hbm, z1_hbm, norm_hbm,
                 deg_sp, cbuf, rbuf, wbuf, drb, dcb, nbuf, dstripe, sem):
    c = lax.axis_index("c")
    s = lax.axis_index("s")
    off = s * STRIPE
    ncf = 2 * PAIRS

    pltpu.sync_copy(z1_hbm.at[pl.ds(off, STRIPE)], dstripe)
    pltpu.sync_copy(dstripe, deg_sp.at[pl.ds(off, STRIPE)])
    plsc.subcore_barrier()

    tbase = s * (CH * ncf)

    def deg_body(i, carry):
        base = tbase + i * CH
        pltpu.sync_copy(col_hbm.at[pl.ds(base, CH)], cbuf)
        pltpu.sync_copy(w_hbm.at[pl.ds(base, CH)], wbuf)
        pltpu.sync_copy(wbuf, deg_sp.at[cbuf], add=True)
        return carry

    lax.fori_loop(0, ncf, deg_body, 0)
    plsc.subcore_barrier()

    pltpu.sync_copy(deg_sp.at[pl.ds(off, STRIPE)], dstripe)

    def dis_body(i, carry):
        d = dstripe[pl.ds(i * 16, 16)]
        dstripe[pl.ds(i * 16, 16)] = _rsqrt_newton(d)
        return carry

    lax.fori_loop(0, STRIPE // 16, dis_body, 0)
    plsc.subcore_barrier()
    pltpu.sync_copy(dstripe, deg_sp.at[pl.ds(off, STRIPE)])
    plsc.subcore_barrier()

    nbase = c * (E_PAD // 2) + s * (CH * (ncf // 2))

    def norm_body(i, carry):
        base = nbase + i * CH
        pltpu.sync_copy(row_hbm.at[pl.ds(base, CH)], rbuf)
        pltpu.sync_copy(col_hbm.at[pl.ds(base, CH)], cbuf)
        pltpu.sync_copy(w_hbm.at[pl.ds(base, CH)], wbuf)
        pltpu.async_copy(deg_sp.at[rbuf], drb, sem).wait()
        pltpu.async_copy(deg_sp.at[cbuf], dcb, sem).wait()

        def m16(g, cc):
            o = g * 16
            nbuf[pl.ds(o, 16)] = (drb[pl.ds(o, 16)] * dcb[pl.ds(o, 16)]
                                  * wbuf[pl.ds(o, 16)])
            return cc

        lax.fori_loop(0, CH // 16, m16, 0)
        pltpu.sync_copy(nbuf, norm_hbm.at[pl.ds(base, CH)])
        return carry

    lax.fori_loop(0, ncf // 2, norm_body, 0)


@functools.partial(
    pl.kernel,
    out_type=[
        jax.ShapeDtypeStruct((NBLK, NP_PAD, FB), jnp.float32),
        jax.ShapeDtypeStruct((NCORES, E_PAD, FB), jnp.float32),
    ],
    mesh=_mesh,
    scratch_types=[
        pltpu.VMEM_SHARED((NP_PAD, FB), jnp.float32),
        pltpu.VMEM((STRIPE, FB), jnp.float32),
        pltpu.VMEM((CH,), jnp.int32),
        pltpu.VMEM((CH,), jnp.int32),
        pltpu.VMEM((CH,), jnp.int32),
        pltpu.VMEM((CH,), jnp.int32),
        pltpu.VMEM((CH,), jnp.float32),
        pltpu.VMEM((CH,), jnp.float32),
        pltpu.VMEM((CH, FB), jnp.float32),
        pltpu.VMEM((CH, FB), jnp.float32),
        pltpu.SemaphoreType.DMA,
        pltpu.SemaphoreType.DMA,
        pltpu.SemaphoreType.DMA,
    ],
)
def _agg_kernel(xb_hbm, row_hbm, col_hbm, norm_hbm, agg_out, ev_hbm,
                sp, stg, ridx0, ridx1, cidx0, cidx1, nrm0, nrm1,
                rows0, rows1, sem0, sem1, semw):
    c = lax.axis_index("c")
    s = lax.axis_index("s")
    off = s * STRIPE
    tbase = s * (2 * CH * PAIRS)
    ridx = (ridx0, ridx1)
    cidx = (cidx0, cidx1)
    nrm = (nrm0, nrm1)
    rows = (rows0, rows1)
    sems = (sem0, sem1)

    for b in range(BPC):
        blk = c * BPC + b

        pltpu.sync_copy(xb_hbm.at[blk].at[pl.ds(off, STRIPE), :], stg)
        pltpu.sync_copy(stg, sp.at[pl.ds(off, STRIPE), :])
        plsc.subcore_barrier()

        def ph1_body(i, carry):
            base = tbase + 2 * i * CH
            for p in range(2):
                pltpu.sync_copy(row_hbm.at[pl.ds(base + p * CH, CH)], ridx[p])
                pltpu.sync_copy(norm_hbm.at[pl.ds(base + p * CH, CH)], nrm[p])
            g0 = pltpu.async_copy(sp.at[ridx[0]], rows[0], sems[0])
            g1 = pltpu.async_copy(sp.at[ridx[1]], rows[1], sems[1])
            for p in range(2):
                (g0 if p == 0 else g1).wait()
                nb = nrm[p]
                rp = rows[p]

                def scale(g, cc):
                    nv = nb[pl.ds(g * 16, 16)]
                    for j in range(16):
                        k = g * 16 + j
                        rp[k, :] = rp[k, :] * nv[j]
                    return cc

                lax.fori_loop(0, CH // 16, scale, 0)
            w0 = pltpu.async_copy(
                rows[0], ev_hbm.at[c].at[pl.ds(base, CH), :], semw)
            w1 = pltpu.async_copy(
                rows[1], ev_hbm.at[c].at[pl.ds(base + CH, CH), :], semw)
            w0.wait()
            w1.wait()
            return carry

        lax.fori_loop(0, PAIRS, ph1_body, 0)
        plsc.subcore_barrier()

        def zero_body(k, carry):
            stg[k, :] = jnp.zeros((FB,), jnp.float32)
            return carry

        lax.fori_loop(0, STRIPE, zero_body, 0)
        pltpu.sync_copy(stg, sp.at[pl.ds(off, STRIPE), :])
        plsc.subcore_barrier()

        def ph2_body(i, carry):
            base = tbase + 2 * i * CH
            for p in range(2):
                pltpu.sync_copy(col_hbm.at[pl.ds(base + p * CH, CH)], cidx[p])
            g0 = pltpu.async_copy(
                ev_hbm.at[c].at[pl.ds(base, CH), :], rows[0], sems[0])
            g1 = pltpu.async_copy(
                ev_hbm.at[c].at[pl.ds(base + CH, CH), :], rows[1], sems[1])
            g0.wait()
            pltpu.sync_copy(rows[0], sp.at[cidx[0]], add=True)
            g1.wait()
            pltpu.sync_copy(rows[1], sp.at[cidx[1]], add=True)
            return carry

        lax.fori_loop(0, PAIRS, ph2_body, 0)
        plsc.subcore_barrier()

        pltpu.sync_copy(sp.at[pl.ds(off, STRIPE), :],
                        agg_out.at[blk].at[pl.ds(off, STRIPE), :])
        plsc.subcore_barrier()


_BN = 2000


def _dense_body(agg_ref, az_ref, ah_ref, cz_ref, chb_ref, probs_ref,
                linw_ref, linb_ref, o_ref):
    a = agg_ref[...]
    az = az_ref[...]
    ah = ah_ref[...]
    cz = cz_ref[...]
    chb = chb_ref[...]
    probs = probs_ref[...]
    acc = jnp.zeros((_BN, F_OUT), jnp.float32)
    for t in range(PERIODS):
        at = a[:, t * F_IN:(t + 1) * F_IN]
        z = jax.nn.sigmoid(
            jnp.dot(at, az, preferred_element_type=jnp.float32) + cz)
        ht = jnp.tanh(
            jnp.dot(at, ah, preferred_element_type=jnp.float32) + chb)
        acc = acc + probs[0, t] * (1.0 - z) * ht
    o_ref[...] = (jnp.dot(jax.nn.relu(acc), linw_ref[...],
                          preferred_element_type=jnp.float32)
                  + linb_ref[...])


def kernel(x, edge_index, edge_weight, Wz, bz, Wr, br, Wh, bh,
           LzW, Lzb, LrW, Lrb, LhW, Lhb, att, linW, linb):
    n = x.shape[0]
    loop = jnp.arange(n, dtype=edge_index.dtype)
    row = jnp.concatenate([edge_index[0], loop])
    col = jnp.concatenate([edge_index[1], loop])
    w = jnp.concatenate([edge_weight, jnp.ones((n,), x.dtype)])
    deg = jax.ops.segment_sum(w, col, num_segments=n)
    dis = lax.rsqrt(deg)
    norm = dis[row] * w * dis[col]
    x96 = (x.transpose(2, 0, 1).reshape(PERIODS, n, F_IN)
           .transpose(1, 0, 2).reshape(n, PERIODS * F_IN))
    agg96 = jax.ops.segment_sum(x96[row] * norm[:, None], col, num_segments=n)

    az = Wz @ LzW[:F_OUT]
    ah = Wh @ LhW[:F_OUT]
    cz = (bz @ LzW[:F_OUT] + Lzb).reshape(1, F_OUT)
    chb = (bh @ LhW[:F_OUT] + Lhb).reshape(1, F_OUT)
    probs = jax.nn.softmax(att).reshape(1, PERIODS)
    linb2 = linb.reshape(1, PERIODS)

    grid = (n // _BN,)
    out = pl.pallas_call(
        _dense_body,
        grid=grid,
        in_specs=[
            pl.BlockSpec((_BN, PERIODS * F_IN), lambda i: (i, 0)),
            pl.BlockSpec((F_IN, F_OUT), lambda i: (0, 0)),
            pl.BlockSpec((F_IN, F_OUT), lambda i: (0, 0)),
            pl.BlockSpec((1, F_OUT), lambda i: (0, 0)),
            pl.BlockSpec((1, F_OUT), lambda i: (0, 0)),
            pl.BlockSpec((1, PERIODS), lambda i: (0, 0)),
            pl.BlockSpec((F_OUT, PERIODS), lambda i: (0, 0)),
            pl.BlockSpec((1, PERIODS), lambda i: (0, 0)),
        ],
        out_specs=pl.BlockSpec((_BN, PERIODS), lambda i: (i, 0)),
        out_shape=jax.ShapeDtypeStruct((n, PERIODS), jnp.float32),
    )(agg96, az, ah, cz, chb, probs, linW, linb2)
    return out


# speedup vs baseline: 22.4210x; 2.7984x over previous
import functools

import jax
import jax.numpy as jnp
from jax import lax
from jax.experimental import pallas as pl
from jax.experimental.pallas import tpu as pltpu
from jax.experimental.pallas import tpu_sc as plsc

N_NODES = 100000
F_IN = 8
F_OUT = 24
PERIODS = 12

NTILES = 16
NCORES = 2
FB = 16
NBLK = PERIODS * F_IN // FB
BPC = NBLK // NCORES

NP_PAD = 100096
STRIPE = NP_PAD // NTILES
CHA = 400
E_PAD = 1702400
NCA = E_PAD // (NTILES * CHA)

CHB = 112
NCB2 = E_PAD // (2 * NTILES * CHB)
G8 = 12544
NCOV = G8 * 8
GST = G8 // NTILES

_mesh = plsc.VectorSubcoreMesh(core_axis_name="c", subcore_axis_name="s")


def _rsqrt_newton(d):
    bits = lax.bitcast_convert_type(d, jnp.int32)
    y = lax.bitcast_convert_type(jnp.int32(0x5F3759DF) - (bits >> 1),
                                 jnp.float32)
    for _ in range(3):
        y = y * (1.5 - 0.5 * d * y * y)
    return y


@functools.partial(
    pl.kernel,
    out_type=jax.ShapeDtypeStruct((E_PAD,), jnp.float32),
    mesh=_mesh,
    scratch_types=[
        pltpu.VMEM_SHARED((NP_PAD,), jnp.float32),
        pltpu.VMEM((CHA,), jnp.int32),
        pltpu.VMEM((CHA,), jnp.int32),
        pltpu.VMEM((CHA,), jnp.float32),
        pltpu.VMEM((CHA,), jnp.float32),
        pltpu.VMEM((CHA,), jnp.float32),
        pltpu.VMEM((CHA,), jnp.float32),
        pltpu.VMEM((STRIPE,), jnp.float32),
        pltpu.SemaphoreType.DMA,
    ],
)
def _norm_kernel(row_hbm, col_hbm, w_hbm, z1_hbm, norm_hbm,
                 deg_sp, cbuf, rbuf, wbuf, drb, dcb, nbuf, dstripe, sem):
    c = lax.axis_index("c")
    s = lax.axis_index("s")
    off = s * STRIPE

    pltpu.sync_copy(z1_hbm.at[pl.ds(off, STRIPE)], dstripe)
    pltpu.sync_copy(dstripe, deg_sp.at[pl.ds(off, STRIPE)])
    plsc.subcore_barrier()

    tbase = s * (CHA * NCA)

    def deg_body(i, carry):
        base = tbase + i * CHA
        pltpu.sync_copy(col_hbm.at[pl.ds(base, CHA)], cbuf)
        pltpu.sync_copy(w_hbm.at[pl.ds(base, CHA)], wbuf)
        pltpu.sync_copy(wbuf, deg_sp.at[cbuf], add=True)
        return carry

    lax.fori_loop(0, NCA, deg_body, 0)
    plsc.subcore_barrier()

    pltpu.sync_copy(deg_sp.at[pl.ds(off, STRIPE)], dstripe)

    def dis_body(i, carry):
        d = dstripe[pl.ds(i * 16, 16)]
        dstripe[pl.ds(i * 16, 16)] = _rsqrt_newton(d)
        return carry

    lax.fori_loop(0, STRIPE // 16, dis_body, 0)
    plsc.subcore_barrier()
    pltpu.sync_copy(dstripe, deg_sp.at[pl.ds(off, STRIPE)])
    plsc.subcore_barrier()

    nbase = c * (E_PAD // 2) + s * (CHA * (NCA // 2))

    def norm_body(i, carry):
        base = nbase + i * CHA
        pltpu.sync_copy(row_hbm.at[pl.ds(base, CHA)], rbuf)
        pltpu.sync_copy(col_hbm.at[pl.ds(base, CHA)], cbuf)
        pltpu.sync_copy(w_hbm.at[pl.ds(base, CHA)], wbuf)
        pltpu.async_copy(deg_sp.at[rbuf], drb, sem).wait()
        pltpu.async_copy(deg_sp.at[cbuf], dcb, sem).wait()

        def m16(g, cc):
            o = g * 16
            nbuf[pl.ds(o, 16)] = (drb[pl.ds(o, 16)] * dcb[pl.ds(o, 16)]
                                  * wbuf[pl.ds(o, 16)])
            return cc

        lax.fori_loop(0, CHA // 16, m16, 0)
        pltpu.sync_copy(nbuf, norm_hbm.at[pl.ds(base, CHA)])
        return carry

    lax.fori_loop(0, NCA // 2, norm_body, 0)


def _make_agg(boff):
    @functools.partial(
        pl.kernel,
        out_type=jax.ShapeDtypeStruct((NCORES, G8, 128), jnp.float32),
        mesh=_mesh,
        scratch_types=[
            pltpu.VMEM_SHARED((G8, 128), jnp.float32),
            pltpu.VMEM((CHB,), jnp.int32),
            pltpu.VMEM((CHB,), jnp.int32),
            pltpu.VMEM((CHB,), jnp.int32),
            pltpu.VMEM((CHB,), jnp.float32),
            pltpu.VMEM((CHB, 128), jnp.float32),
            pltpu.VMEM((CHB, 128), jnp.float32),
            pltpu.SemaphoreType.DMA,
        ],
    )
    def agg(xpad_hbm, row_hbm, col_hbm, norm_hbm, agg_out,
            sp, ridx, cidx, cidx8, nrm, rows, scat, sem):
        c = lax.axis_index("c")
        s = lax.axis_index("s")
        tbase = c * (E_PAD // 2) + s * (CHB * NCB2)
        z16 = jnp.zeros((16,), jnp.float32)

        def zs(k, cc):
            for j in range(8):
                scat[k, pl.ds(j * 16, 16)] = z16
            return cc

        lax.fori_loop(0, CHB, zs, 0)

        for i in range(GST // CHB):
            pltpu.sync_copy(scat, sp.at[pl.ds(s * GST + i * CHB, CHB), :])
        plsc.subcore_barrier()

        def chunk(i, cc):
            base = tbase + i * CHB
            pltpu.sync_copy(row_hbm.at[pl.ds(base, CHB)], ridx)
            pltpu.sync_copy(col_hbm.at[pl.ds(base, CHB)], cidx)
            pltpu.sync_copy(norm_hbm.at[pl.ds(base, CHB)], nrm)
            pltpu.async_copy(xpad_hbm.at[ridx], rows, sem).wait()

            def grp(g, cc2):
                o = g * 16
                cv = cidx[pl.ds(o, 16)]
                cidx8[pl.ds(o, 16)] = cv >> 3
                sub16 = (cv & 7) * 16
                nv = nrm[pl.ds(o, 16)]
                for j in range(16):
                    k = o + j
                    v = rows[k, pl.ds(boff, 16)] * nv[j]
                    scat[k, pl.ds(sub16[j], 16)] = v
                return cc2

            lax.fori_loop(0, CHB // 16, grp, 0)
            pltpu.sync_copy(scat, sp.at[cidx8], add=True)

            def rz(g, cc2):
                o = g * 16
                sub16 = (cidx[pl.ds(o, 16)] & 7) * 16
                for j in range(16):
                    scat[o + j, pl.ds(sub16[j], 16)] = z16
                return cc2

            lax.fori_loop(0, CHB // 16, rz, 0)
            return cc

        lax.fori_loop(0, NCB2, chunk, 0)
        plsc.subcore_barrier()

        @pl.when(c == 0)
        def _():
            pltpu.sync_copy(sp.at[pl.ds(s * GST, GST), :],
                            agg_out.at[0].at[pl.ds(s * GST, GST), :])

        @pl.when(c == 1)
        def _():
            pltpu.sync_copy(sp.at[pl.ds(s * GST, GST), :],
                            agg_out.at[1].at[pl.ds(s * GST, GST), :])

    return agg


_agg_kernels = [_make_agg(p * FB) for p in range(NBLK)]


_BN = 2000


def _dense_body(agg_ref, az_ref, ah_ref, cz_ref, chb_ref, probs_ref,
                linw_ref, linb_ref, o_ref):
    a = agg_ref[...]
    az = az_ref[...]
    ah = ah_ref[...]
    cz = cz_ref[...]
    chb = chb_ref[...]
    probs = probs_ref[...]
    acc = jnp.zeros((_BN, F_OUT), jnp.float32)
    for t in range(PERIODS):
        at = a[:, t * F_IN:(t + 1) * F_IN]
        z = jax.nn.sigmoid(
            jnp.dot(at, az, preferred_element_type=jnp.float32) + cz)
        ht = jnp.tanh(
            jnp.dot(at, ah, preferred_element_type=jnp.float32) + chb)
        acc = acc + probs[0, t] * (1.0 - z) * ht
    o_ref[...] = (jnp.dot(jax.nn.relu(acc), linw_ref[...],
                          preferred_element_type=jnp.float32)
                  + linb_ref[...])


def kernel(x, edge_index, edge_weight, Wz, bz, Wr, br, Wh, bh,
           LzW, Lzb, LrW, Lrb, LhW, Lhb, att, linW, linb):
    n = x.shape[0]
    e = edge_index.shape[1]

    loop = jnp.arange(n, dtype=edge_index.dtype)
    row = jnp.concatenate([edge_index[0], loop])
    col = jnp.concatenate([edge_index[1], loop])
    w = jnp.concatenate([edge_weight, jnp.ones((n,), x.dtype)])
    pad = E_PAD - (e + n)
    row = jnp.concatenate([row, jnp.zeros((pad,), row.dtype)])
    col = jnp.concatenate([col, jnp.zeros((pad,), col.dtype)])
    w = jnp.concatenate([w, jnp.zeros((pad,), w.dtype)])

    x96 = (x.transpose(2, 0, 1).reshape(PERIODS, n, F_IN)
           .transpose(1, 0, 2).reshape(n, PERIODS * F_IN))
    xpad = jnp.pad(x96, ((0, 0), (0, 128 - PERIODS * F_IN)))

    z1 = jnp.zeros((NP_PAD,), jnp.float32)

    norm = _norm_kernel(row, col, w, z1)

    parts = [k(xpad, row, col, norm) for k in _agg_kernels]
    agg_out = jnp.stack([p[0] + p[1] for p in parts])

    agg96 = (agg_out.reshape(NBLK, NCOV, FB)[:, :n, :]
             .transpose(1, 0, 2).reshape(n, PERIODS * F_IN))

    az = Wz @ LzW[:F_OUT]
    ah = Wh @ LhW[:F_OUT]
    cz = (bz @ LzW[:F_OUT] + Lzb).reshape(1, F_OUT)
    chb = (bh @ LhW[:F_OUT] + Lhb).reshape(1, F_OUT)
    probs = jax.nn.softmax(att).reshape(1, PERIODS)
    linb2 = linb.reshape(1, PERIODS)

    grid = (n // _BN,)
    out = pl.pallas_call(
        _dense_body,
        grid=grid,
        in_specs=[
            pl.BlockSpec((_BN, PERIODS * F_IN), lambda i: (i, 0)),
            pl.BlockSpec((F_IN, F_OUT), lambda i: (0, 0)),
            pl.BlockSpec((F_IN, F_OUT), lambda i: (0, 0)),
            pl.BlockSpec((1, F_OUT), lambda i: (0, 0)),
            pl.BlockSpec((1, F_OUT), lambda i: (0, 0)),
            pl.BlockSpec((1, PERIODS), lambda i: (0, 0)),
            pl.BlockSpec((F_OUT, PERIODS), lambda i: (0, 0)),
            pl.BlockSpec((1, PERIODS), lambda i: (0, 0)),
        ],
        out_specs=pl.BlockSpec((_BN, PERIODS), lambda i: (i, 0)),
        out_shape=jax.ShapeDtypeStruct((n, PERIODS), jnp.float32),
    )(agg96, az, ah, cz, chb, probs, linW, linb2)
    return out


# speedup vs baseline: 27.3772x; 1.2211x over previous
import functools

import jax
import jax.numpy as jnp
from jax import lax
from jax.experimental import pallas as pl
from jax.experimental.pallas import tpu as pltpu
from jax.experimental.pallas import tpu_sc as plsc

N_NODES = 100000
F_IN = 8
F_OUT = 24
PERIODS = 12

NTILES = 16
NCORES = 2
FB = 16
NBLK = PERIODS * F_IN // FB
BPC = NBLK // NCORES

NP_PAD = 100096
STRIPE = NP_PAD // NTILES
CHA = 400
E_PAD = 1702400
NCA = E_PAD // (NTILES * CHA)

CHB = 112
NCB2 = E_PAD // (2 * NTILES * CHB)
G8 = 12544
NCOV = G8 * 8
GST = G8 // NTILES

_mesh = plsc.VectorSubcoreMesh(core_axis_name="c", subcore_axis_name="s")


def _rsqrt_newton(d):
    bits = lax.bitcast_convert_type(d, jnp.int32)
    y = lax.bitcast_convert_type(jnp.int32(0x5F3759DF) - (bits >> 1),
                                 jnp.float32)
    for _ in range(3):
        y = y * (1.5 - 0.5 * d * y * y)
    return y


@functools.partial(
    pl.kernel,
    out_type=jax.ShapeDtypeStruct((E_PAD,), jnp.float32),
    mesh=_mesh,
    scratch_types=[
        pltpu.VMEM_SHARED((NP_PAD,), jnp.float32),
        pltpu.VMEM((CHA,), jnp.int32),
        pltpu.VMEM((CHA,), jnp.int32),
        pltpu.VMEM((CHA,), jnp.float32),
        pltpu.VMEM((CHA,), jnp.float32),
        pltpu.VMEM((CHA,), jnp.float32),
        pltpu.VMEM((CHA,), jnp.float32),
        pltpu.VMEM((STRIPE,), jnp.float32),
        pltpu.SemaphoreType.DMA,
    ],
)
def _norm_kernel(row_hbm, col_hbm, w_hbm, z1_hbm, norm_hbm,
                 deg_sp, cbuf, rbuf, wbuf, drb, dcb, nbuf, dstripe, sem):
    c = lax.axis_index("c")
    s = lax.axis_index("s")
    off = s * STRIPE

    pltpu.sync_copy(z1_hbm.at[pl.ds(off, STRIPE)], dstripe)
    pltpu.sync_copy(dstripe, deg_sp.at[pl.ds(off, STRIPE)])
    plsc.subcore_barrier()

    tbase = s * (CHA * NCA)

    def deg_body(i, carry):
        base = tbase + i * CHA
        pltpu.sync_copy(col_hbm.at[pl.ds(base, CHA)], cbuf)
        pltpu.sync_copy(w_hbm.at[pl.ds(base, CHA)], wbuf)
        pltpu.sync_copy(wbuf, deg_sp.at[cbuf], add=True)
        return carry

    lax.fori_loop(0, NCA, deg_body, 0)
    plsc.subcore_barrier()

    pltpu.sync_copy(deg_sp.at[pl.ds(off, STRIPE)], dstripe)

    def dis_body(i, carry):
        d = dstripe[pl.ds(i * 16, 16)]
        dstripe[pl.ds(i * 16, 16)] = _rsqrt_newton(d)
        return carry

    lax.fori_loop(0, STRIPE // 16, dis_body, 0)
    plsc.subcore_barrier()
    pltpu.sync_copy(dstripe, deg_sp.at[pl.ds(off, STRIPE)])
    plsc.subcore_barrier()

    nbase = c * (E_PAD // 2) + s * (CHA * (NCA // 2))

    def norm_body(i, carry):
        base = nbase + i * CHA
        pltpu.sync_copy(row_hbm.at[pl.ds(base, CHA)], rbuf)
        pltpu.sync_copy(col_hbm.at[pl.ds(base, CHA)], cbuf)
        pltpu.sync_copy(w_hbm.at[pl.ds(base, CHA)], wbuf)
        pltpu.async_copy(deg_sp.at[rbuf], drb, sem).wait()
        pltpu.async_copy(deg_sp.at[cbuf], dcb, sem).wait()

        def m16(g, cc):
            o = g * 16
            nbuf[pl.ds(o, 16)] = (drb[pl.ds(o, 16)] * dcb[pl.ds(o, 16)]
                                  * wbuf[pl.ds(o, 16)])
            return cc

        lax.fori_loop(0, CHA // 16, m16, 0)
        pltpu.sync_copy(nbuf, norm_hbm.at[pl.ds(base, CHA)])
        return carry

    lax.fori_loop(0, NCA // 2, norm_body, 0)


def _make_agg(boff):
    @functools.partial(
        pl.kernel,
        out_type=jax.ShapeDtypeStruct((NCORES, G8, 128), jnp.float32),
        mesh=_mesh,
        scratch_types=[
            pltpu.VMEM_SHARED((G8, 128), jnp.float32),
            pltpu.VMEM((3, 128), jnp.int32),
            pltpu.VMEM((CHB,), jnp.int32),
            pltpu.VMEM((CHB, 128), jnp.float32),
            pltpu.VMEM((CHB, 128), jnp.float32),
            pltpu.SemaphoreType.DMA,
        ],
    )
    def agg(xpad_hbm, edata_hbm, agg_out,
            sp, ebuf, cidx8, rows, scat, sem):
        c = lax.axis_index("c")
        s = lax.axis_index("s")
        z16 = jnp.zeros((16,), jnp.float32)

        def zs(k, cc):
            for j in range(8):
                scat[k, pl.ds(j * 16, 16)] = z16
            return cc

        lax.fori_loop(0, CHB, zs, 0)

        for i in range(GST // CHB):
            pltpu.sync_copy(scat, sp.at[pl.ds(s * GST + i * CHB, CHB), :])
        plsc.subcore_barrier()

        tchunk = c * (E_PAD // 2 // CHB) + s * NCB2

        def chunk(i, cc):
            pltpu.sync_copy(edata_hbm.at[tchunk + i], ebuf)
            pltpu.async_copy(xpad_hbm.at[ebuf.at[0].at[pl.ds(0, CHB)]],
                             rows, sem).wait()

            def grp(g, cc2):
                o = g * 16
                cv = ebuf[1, pl.ds(o, 16)]
                cidx8[pl.ds(o, 16)] = cv >> 3
                sub16 = (cv & 7) * 16
                nv = lax.bitcast_convert_type(ebuf[2, pl.ds(o, 16)],
                                              jnp.float32)
                for j in range(16):
                    k = o + j
                    v = rows[k, pl.ds(boff, 16)] * nv[j]
                    scat[k, pl.ds(sub16[j], 16)] = v
                return cc2

            lax.fori_loop(0, CHB // 16, grp, 0)
            pltpu.sync_copy(scat, sp.at[cidx8], add=True)

            def rz(g, cc2):
                o = g * 16
                sub16 = (ebuf[1, pl.ds(o, 16)] & 7) * 16
                for j in range(16):
                    scat[o + j, pl.ds(sub16[j], 16)] = z16
                return cc2

            lax.fori_loop(0, CHB // 16, rz, 0)
            return cc

        lax.fori_loop(0, NCB2, chunk, 0)
        plsc.subcore_barrier()

        @pl.when(c == 0)
        def _():
            pltpu.sync_copy(sp.at[pl.ds(s * GST, GST), :],
                            agg_out.at[0].at[pl.ds(s * GST, GST), :])

        @pl.when(c == 1)
        def _():
            pltpu.sync_copy(sp.at[pl.ds(s * GST, GST), :],
                            agg_out.at[1].at[pl.ds(s * GST, GST), :])

    return agg


_agg_kernels = [_make_agg(p * FB) for p in range(NBLK)]


_BN = 2000


def _dense_body(agg_ref, az_ref, ah_ref, cz_ref, chb_ref, probs_ref,
                linw_ref, linb_ref, o_ref):
    a = agg_ref[...]
    az = az_ref[...]
    ah = ah_ref[...]
    cz = cz_ref[...]
    chb = chb_ref[...]
    probs = probs_ref[...]
    acc = jnp.zeros((_BN, F_OUT), jnp.float32)
    for t in range(PERIODS):
        at = a[:, t * F_IN:(t + 1) * F_IN]
        z = jax.nn.sigmoid(
            jnp.dot(at, az, preferred_element_type=jnp.float32) + cz)
        ht = jnp.tanh(
            jnp.dot(at, ah, preferred_element_type=jnp.float32) + chb)
        acc = acc + probs[0, t] * (1.0 - z) * ht
    o_ref[...] = (jnp.dot(jax.nn.relu(acc), linw_ref[...],
                          preferred_element_type=jnp.float32)
                  + linb_ref[...])


def kernel(x, edge_index, edge_weight, Wz, bz, Wr, br, Wh, bh,
           LzW, Lzb, LrW, Lrb, LhW, Lhb, att, linW, linb):
    n = x.shape[0]
    e = edge_index.shape[1]

    loop = jnp.arange(n, dtype=edge_index.dtype)
    row = jnp.concatenate([edge_index[0], loop])
    col = jnp.concatenate([edge_index[1], loop])
    w = jnp.concatenate([edge_weight, jnp.ones((n,), x.dtype)])
    pad = E_PAD - (e + n)
    row = jnp.concatenate([row, jnp.zeros((pad,), row.dtype)])
    col = jnp.concatenate([col, jnp.zeros((pad,), col.dtype)])
    w = jnp.concatenate([w, jnp.zeros((pad,), w.dtype)])

    x96 = (x.transpose(2, 0, 1).reshape(PERIODS, n, F_IN)
           .transpose(1, 0, 2).reshape(n, PERIODS * F_IN))
    xpad = jnp.pad(x96, ((0, 0), (0, 128 - PERIODS * F_IN)))

    z1 = jnp.zeros((NP_PAD,), jnp.float32)

    norm = _norm_kernel(row, col, w, z1)

    edata = (jnp.stack([row, col, lax.bitcast_convert_type(norm, jnp.int32)])
             .reshape(3, E_PAD // CHB, CHB).transpose(1, 0, 2))
    edata = jnp.pad(edata, ((0, 0), (0, 0), (0, 128 - CHB)))

    parts = [k(xpad, edata) for k in _agg_kernels]
    agg_out = jnp.stack([p[0] + p[1] for p in parts])

    agg96 = (agg_out.reshape(NBLK, NCOV, FB)[:, :n, :]
             .transpose(1, 0, 2).reshape(n, PERIODS * F_IN))

    az = Wz @ LzW[:F_OUT]
    ah = Wh @ LhW[:F_OUT]
    cz = (bz @ LzW[:F_OUT] + Lzb).reshape(1, F_OUT)
    chb = (bh @ LhW[:F_OUT] + Lhb).reshape(1, F_OUT)
    probs = jax.nn.softmax(att).reshape(1, PERIODS)
    linb2 = linb.reshape(1, PERIODS)

    grid = (n // _BN,)
    out = pl.pallas_call(
        _dense_body,
        grid=grid,
        in_specs=[
            pl.BlockSpec((_BN, PERIODS * F_IN), lambda i: (i, 0)),
            pl.BlockSpec((F_IN, F_OUT), lambda i: (0, 0)),
            pl.BlockSpec((F_IN, F_OUT), lambda i: (0, 0)),
            pl.BlockSpec((1, F_OUT), lambda i: (0, 0)),
            pl.BlockSpec((1, F_OUT), lambda i: (0, 0)),
            pl.BlockSpec((1, PERIODS), lambda i: (0, 0)),
            pl.BlockSpec((F_OUT, PERIODS), lambda i: (0, 0)),
            pl.BlockSpec((1, PERIODS), lambda i: (0, 0)),
        ],
        out_specs=pl.BlockSpec((_BN, PERIODS), lambda i: (i, 0)),
        out_shape=jax.ShapeDtypeStruct((n, PERIODS), jnp.float32),
    )(agg96, az, ah, cz, chb, probs, linW, linb2)
    return out
